# Initial kernel scaffold; baseline (speedup 1.0000x reference)
import jax
import jax.numpy as jnp
from jax.experimental import pallas as pl


def kernel(x, edge_index, edge_features, params):
    raise NotImplementedError("write your pallas kernel here")


# speedup vs baseline: 1.9311x; 1.9311x over previous
import functools

import jax
import jax.numpy as jnp
from jax import lax
from jax.experimental import pallas as pl
from jax.experimental.pallas import tpu as pltpu
from jax.experimental.pallas import tpu_sc as plsc

F32 = jnp.float32

N_NODES = 10000
N_EDGES = 160000
LATENT = 256

BN = 2000
BE = 2000

NC = 2
NS = 16
NW = NC * NS

CH = 128
NCHUNK = N_EDGES // CH
SK = 200
ACCR = N_NODES


def _dot(a, w):
    return lax.dot_general(a, w, (((1,), (0,)), ((), ())),
                           preferred_element_type=F32)


def _ln(y, g, b, eps=1e-5):
    m = jnp.mean(y, axis=-1, keepdims=True)
    v = jnp.var(y, axis=-1, keepdims=True)
    return (y - m) / jnp.sqrt(v + eps) * g + b


def _row_spec(block, ncols):
    return pl.BlockSpec((block, ncols), lambda i: (i, 0))


def _full_spec(shape):
    nd = len(shape)
    return pl.BlockSpec(shape, lambda i: (0,) * nd)


def _node_enc_body(x, w0, b0, w1, b1, w2, b2, g, be, wa, wb,
                   h_ref, ha_ref, hb_ref):
    t = jnp.maximum(_dot(x[...], w0[...]) + b0[...], 0.0)
    t = jnp.maximum(_dot(t, w1[...]) + b1[...], 0.0)
    h = _ln(_dot(t, w2[...]) + b2[...], g[...], be[...])
    h_ref[...] = h
    ha_ref[...] = _dot(h, wa[...])
    hb_ref[...] = _dot(h, wb[...])


def _edge_enc_body(x, w0, b0, w1, b1, w2, b2, g, be, e_ref):
    t = jnp.maximum(_dot(x[...], w0[...]) + b0[...], 0.0)
    t = jnp.maximum(_dot(t, w1[...]) + b1[...], 0.0)
    e_ref[...] = _ln(_dot(t, w2[...]) + b2[...], g[...], be[...])


def _edge_step_body(xi, xj, e, w1c, b0, w1, b1, w2, b2, g, be,
                    msg_ref, eo_ref):
    t = xi[...] + xj[...] + _dot(e[...], w1c[...]) + b0[...]
    t = jnp.maximum(t, 0.0)
    t = jnp.maximum(_dot(t, w1[...]) + b1[...], 0.0)
    m = _ln(_dot(t, w2[...]) + b2[...], g[...], be[...])
    msg_ref[...] = m
    eo_ref[...] = e[...] + m


def _node_step_body(agg, h, wa, wb, b0, w1, b1, w2, b2, g, be, ewa, ewb,
                    hn_ref, ha_ref, hb_ref):
    t = jnp.maximum(_dot(agg[...].astype(F32), wa[...])
                    + _dot(h[...], wb[...]) + b0[...], 0.0)
    t = jnp.maximum(_dot(t, w1[...]) + b1[...], 0.0)
    hn = h[...] + _ln(_dot(t, w2[...]) + b2[...], g[...], be[...])
    hn_ref[...] = hn
    ha_ref[...] = _dot(hn, ewa[...])
    hb_ref[...] = _dot(hn, ewb[...])


def _node_last_body(agg, h, wa, wb, b0, w1, b1, w2, b2, g, be, hn_ref):
    t = jnp.maximum(_dot(agg[...].astype(F32), wa[...])
                    + _dot(h[...], wb[...]) + b0[...], 0.0)
    t = jnp.maximum(_dot(t, w1[...]) + b1[...], 0.0)
    hn_ref[...] = h[...] + _ln(_dot(t, w2[...]) + b2[...], g[...], be[...])


def _dec_body(h, w0, b0, w1, b1, w2, b2, out_ref):
    t = jnp.maximum(_dot(h[...], w0[...]) + b0[...], 0.0)
    t = jnp.maximum(_dot(t, w1[...]) + b1[...], 0.0)
    out_ref[...] = _dot(t, w2[...]) + b2[...]


def _tc_call(body, nrows, block, in_specs_cols, weight_shapes, out_cols,
             n_out, out_dtypes=None):
    grid = nrows // block
    in_specs = ([_row_spec(block, c) for c in in_specs_cols]
                + [_full_spec(s) for s in weight_shapes])
    out_specs = [_row_spec(block, c) for c in out_cols]
    if out_dtypes is None:
        out_dtypes = [F32] * len(out_cols)
    out_shape = [jax.ShapeDtypeStruct((nrows, c), d)
                 for c, d in zip(out_cols, out_dtypes)]
    if n_out == 1:
        out_specs, out_shape = out_specs[0], out_shape[0]
    return pl.pallas_call(
        body,
        grid=(grid,),
        in_specs=in_specs,
        out_specs=out_specs,
        out_shape=out_shape,
        compiler_params=pltpu.CompilerParams(
            dimension_semantics=("arbitrary",)),
    )


_HALF = LATENT // NC
_NPS = N_NODES // NS


@functools.lru_cache(maxsize=None)
def _build_sc_gather():
    mesh = plsc.VectorSubcoreMesh(core_axis_name="c", subcore_axis_name="s")

    nk = (NCHUNK + NW - 1) // NW

    @functools.partial(
        pl.kernel,
        out_type=(jax.ShapeDtypeStruct((N_EDGES, LATENT), F32),
                  jax.ShapeDtypeStruct((N_EDGES, LATENT), F32)),
        mesh=mesh,
        scratch_types=[
            pltpu.VMEM((CH,), jnp.int32),
            pltpu.VMEM((CH,), jnp.int32),
            pltpu.VMEM((CH, LATENT), F32),
            pltpu.VMEM((CH, LATENT), F32),
            pltpu.SemaphoreType.DMA,
        ],
    )
    def sc_gather(ha_hbm, hb_hbm, dst_hbm, src_hbm, xi_hbm, xj_hbm,
                  idxd_v, idxs_v, bufa_v, bufb_v, sem):
        wid = lax.axis_index("s") * NC + lax.axis_index("c")

        def chunk(k, carry):
            c = k * NW + wid

            @pl.when(c < NCHUNK)
            def _():
                base = c * CH
                pltpu.sync_copy(dst_hbm.at[pl.ds(base, CH)], idxd_v)
                pltpu.sync_copy(src_hbm.at[pl.ds(base, CH)], idxs_v)
                a = pltpu.async_copy(ha_hbm.at[idxd_v], bufa_v, sem)
                b = pltpu.async_copy(hb_hbm.at[idxs_v], bufb_v, sem)
                a.wait()
                b.wait()
                pltpu.sync_copy(bufa_v, xi_hbm.at[pl.ds(base, CH)])
                pltpu.sync_copy(bufb_v, xj_hbm.at[pl.ds(base, CH)])
            return carry

        lax.fori_loop(0, nk, chunk, 0)

    return sc_gather


@functools.lru_cache(maxsize=None)
def _build_sc_permute():
    mesh = plsc.VectorSubcoreMesh(core_axis_name="c", subcore_axis_name="s")

    nk = (NCHUNK + NW - 1) // NW

    @functools.partial(
        pl.kernel,
        out_type=jax.ShapeDtypeStruct((N_EDGES, LATENT), F32),
        mesh=mesh,
        scratch_types=[
            pltpu.VMEM((CH,), jnp.int32),
            pltpu.VMEM((CH, LATENT), F32),
            pltpu.SemaphoreType.DMA,
        ],
    )
    def sc_permute(tab_hbm, perm_hbm, out_hbm, idx_v, buf_v, sem):
        wid = lax.axis_index("s") * NC + lax.axis_index("c")

        def chunk(k, carry):
            c = k * NW + wid

            @pl.when(c < NCHUNK)
            def _():
                base = c * CH
                pltpu.sync_copy(perm_hbm.at[pl.ds(base, CH)], idx_v)
                pltpu.async_copy(tab_hbm.at[idx_v], buf_v, sem).wait()
                pltpu.sync_copy(buf_v, out_hbm.at[pl.ds(base, CH)])
            return carry

        lax.fori_loop(0, nk, chunk, 0)

    return sc_permute


_CPB = NCHUNK // NS
_CREM = NCHUNK - _CPB * NS


@functools.lru_cache(maxsize=None)
def _build_sc_scatter():
    mesh = plsc.VectorSubcoreMesh(core_axis_name="c", subcore_axis_name="s")

    nzc = ACCR // SK

    @functools.partial(
        pl.kernel,
        out_type=jax.ShapeDtypeStruct((N_NODES, LATENT), F32),
        mesh=mesh,
        scratch_types=[
            pltpu.VMEM((CH,), jnp.int32),
            pltpu.VMEM((CH, _HALF), F32),
            pltpu.VMEM_SHARED((ACCR, _HALF), F32),
        ],
    )
    def sc_scatter(msg_hbm, dst_hbm, zeros_hbm, agg_hbm,
                   idx_v, rows_v, acc_sh):
        cid = lax.axis_index("c")
        sid = lax.axis_index("s")
        col0 = cid * _HALF
        for k in range((nzc + NS - 1) // NS):
            c = sid + k * NS

            @pl.when(c < nzc)
            def _():
                pltpu.sync_copy(zeros_hbm, acc_sh.at[pl.ds(c * SK, SK)])
        plsc.subcore_barrier()

        cbase = sid * _CPB + jnp.minimum(sid, _CREM)
        ccnt = _CPB + jnp.where(sid < _CREM, 1, 0)

        def chunk(k, carry):
            @pl.when(k < ccnt)
            def _():
                base = (cbase + k) * CH
                pltpu.sync_copy(dst_hbm.at[pl.ds(base, CH)], idx_v)
                pltpu.sync_copy(
                    msg_hbm.at[pl.ds(base, CH), pl.ds(col0, _HALF)],
                    rows_v)
                pltpu.sync_copy(rows_v, acc_sh.at[idx_v], add=True)
            return carry

        lax.fori_loop(0, _CPB + (1 if _CREM else 0), chunk, 0)
        plsc.subcore_barrier()
        for k in range((nzc + NS - 1) // NS):
            c = sid + k * NS

            @pl.when(c < nzc)
            def _():
                pltpu.sync_copy(
                    acc_sh.at[pl.ds(c * SK, SK)],
                    agg_hbm.at[pl.ds(c * SK, SK), pl.ds(col0, _HALF)])

    return sc_scatter


def _sc_gather(ha, hb, dst, src):
    return _build_sc_gather()(ha, hb, dst, src)


def _sc_permute(tab, perm):
    return _build_sc_permute()(tab, perm)


def _sc_scatter(msg, dst, zeros):
    return _build_sc_scatter()(msg, dst, zeros)


def _prep_mlp(ps):
    out = []
    for W, b in ps:
        out.append(W)
        out.append(b.reshape(1, -1))
    return out


def kernel(x, edge_index, edge_features, params):
    idx = edge_index.astype(jnp.int32)
    zeros = jnp.zeros((SK, _HALF), F32)
    perm = jnp.argsort(idx[1], stable=True).astype(jnp.int32)
    src = idx[0][perm]
    dst = idx[1][perm]

    blocks = params["blocks"]
    eW1 = [blk["edge_mlp"][0][0] for blk in blocks]
    eW1a = [w[:LATENT] for w in eW1]
    eW1b = [w[LATENT:2 * LATENT] for w in eW1]
    eW1c = [w[2 * LATENT:] for w in eW1]
    nW1 = [blk["node_mlp"][0][0] for blk in blocks]
    nW1a = [w[:LATENT] for w in nW1]
    nW1b = [w[LATENT:] for w in nW1]

    WS = (LATENT, LATENT)
    BS1 = (1, LATENT)

    ne = _prep_mlp(params["node_enc"])
    g, be = params["node_enc_ln"]
    node_enc = _tc_call(
        _node_enc_body, N_NODES, BN, [x.shape[1]],
        [(x.shape[1], LATENT), BS1, WS, BS1, WS, BS1, BS1, BS1, WS, WS],
        [LATENT, LATENT, LATENT], 3)
    h, ha, hb = node_enc(x, ne[0], ne[1], ne[2], ne[3], ne[4], ne[5],
                         g.reshape(1, -1), be.reshape(1, -1),
                         eW1a[0], eW1b[0])

    ee = _prep_mlp(params["edge_enc"])
    ge, bee = params["edge_enc_ln"]
    nin_e = edge_features.shape[1]
    edge_enc = _tc_call(
        _edge_enc_body, N_EDGES, BE, [nin_e],
        [(nin_e, LATENT), BS1, WS, BS1, WS, BS1, BS1, BS1],
        [LATENT], 1)
    e = edge_enc(edge_features, ee[0], ee[1], ee[2], ee[3], ee[4], ee[5],
                 ge.reshape(1, -1), bee.reshape(1, -1))
    e = _sc_permute(e, perm)

    edge_step = _tc_call(
        _edge_step_body, N_EDGES, BE, [LATENT, LATENT, LATENT],
        [WS, BS1, WS, BS1, WS, BS1, BS1, BS1],
        [LATENT, LATENT], 2)
    node_step = _tc_call(
        _node_step_body, N_NODES, BN, [LATENT, LATENT],
        [WS, WS, BS1, WS, BS1, WS, BS1, BS1, BS1, WS, WS],
        [LATENT, LATENT, LATENT], 3)
    node_last = _tc_call(
        _node_last_body, N_NODES, BN, [LATENT, LATENT],
        [WS, WS, BS1, WS, BS1, WS, BS1, BS1, BS1],
        [LATENT], 1)

    nsteps = len(blocks)
    for s in range(nsteps):
        blk = blocks[s]
        em = _prep_mlp(blk["edge_mlp"])
        nm = _prep_mlp(blk["node_mlp"])
        eg, eb = blk["edge_ln"]
        ng, nb = blk["node_ln"]

        xi, xj = _sc_gather(ha, hb, dst, src)
        msg, e = edge_step(xi, xj, e, eW1c[s], em[1], em[2], em[3],
                           em[4], em[5], eg.reshape(1, -1), eb.reshape(1, -1))
        agg = _sc_scatter(msg, dst, zeros)
        if s + 1 < nsteps:
            h, ha, hb = node_step(agg, h, nW1a[s], nW1b[s], nm[1], nm[2],
                                  nm[3], nm[4], nm[5], ng.reshape(1, -1),
                                  nb.reshape(1, -1), eW1a[s + 1], eW1b[s + 1])
        else:
            h = node_last(agg, h, nW1a[s], nW1b[s], nm[1], nm[2], nm[3],
                          nm[4], nm[5], ng.reshape(1, -1), nb.reshape(1, -1))

    dm = _prep_mlp(params["dec"])
    nout = dm[4].shape[1]
    w2p = jnp.zeros((LATENT, 128), F32).at[:, :nout].set(dm[4])
    b2p = jnp.zeros((1, 128), F32).at[:, :nout].set(dm[5])
    dec = _tc_call(
        _dec_body, N_NODES, BN, [LATENT],
        [WS, BS1, WS, BS1, (LATENT, 128), (1, 128)],
        [128], 1)
    out = dec(h, dm[0], dm[1], dm[2], dm[3], w2p, b2p)
    return out[:, :nout]


# speedup vs baseline: 1.9328x; 1.0009x over previous
import functools

import jax
import jax.numpy as jnp
from jax import lax
from jax.experimental import pallas as pl
from jax.experimental.pallas import tpu as pltpu
from jax.experimental.pallas import tpu_sc as plsc

F32 = jnp.float32

N_NODES = 10000
N_EDGES = 160000
LATENT = 256

BN = 2000
BE = 2000

NC = 2
NS = 16
NW = NC * NS

CH = 128
NCHUNK = N_EDGES // CH
SK = 200
ACCR = N_NODES


def _dot(a, w):
    return lax.dot_general(a, w, (((1,), (0,)), ((), ())),
                           preferred_element_type=F32)


def _ln(y, g, b, eps=1e-5):
    m = jnp.mean(y, axis=-1, keepdims=True)
    v = jnp.var(y, axis=-1, keepdims=True)
    return (y - m) / jnp.sqrt(v + eps) * g + b


def _row_spec(block, ncols):
    return pl.BlockSpec((block, ncols), lambda i: (i, 0))


def _full_spec(shape):
    nd = len(shape)
    return pl.BlockSpec(shape, lambda i: (0,) * nd)


def _node_enc_body(x, w0, b0, w1, b1, w2, b2, g, be, wa, wb,
                   h_ref, ha_ref, hb_ref):
    t = jnp.maximum(_dot(x[...], w0[...]) + b0[...], 0.0)
    t = jnp.maximum(_dot(t, w1[...]) + b1[...], 0.0)
    h = _ln(_dot(t, w2[...]) + b2[...], g[...], be[...])
    h_ref[...] = h
    ha_ref[...] = _dot(h, wa[...])
    hb_ref[...] = _dot(h, wb[...])


def _edge_enc_body(x, w0, b0, w1, b1, w2, b2, g, be, e_ref):
    t = jnp.maximum(_dot(x[...], w0[...]) + b0[...], 0.0)
    t = jnp.maximum(_dot(t, w1[...]) + b1[...], 0.0)
    e_ref[...] = _ln(_dot(t, w2[...]) + b2[...], g[...], be[...])


def _edge_step_body(xi, xj, e, w1c, b0, w1, b1, w2, b2, g, be,
                    msg_ref, eo_ref):
    t = xi[...] + xj[...] + _dot(e[...], w1c[...]) + b0[...]
    t = jnp.maximum(t, 0.0)
    t = jnp.maximum(_dot(t, w1[...]) + b1[...], 0.0)
    m = _ln(_dot(t, w2[...]) + b2[...], g[...], be[...])
    msg_ref[...] = m
    eo_ref[...] = e[...] + m


def _node_step_body(agg, h, wa, wb, b0, w1, b1, w2, b2, g, be, ewa, ewb,
                    hn_ref, ha_ref, hb_ref):
    t = jnp.maximum(_dot(agg[...], wa[...])
                    + _dot(h[...], wb[...]) + b0[...], 0.0)
    t = jnp.maximum(_dot(t, w1[...]) + b1[...], 0.0)
    hn = h[...] + _ln(_dot(t, w2[...]) + b2[...], g[...], be[...])
    hn_ref[...] = hn
    ha_ref[...] = _dot(hn, ewa[...])
    hb_ref[...] = _dot(hn, ewb[...])


def _node_last_body(agg, h, wa, wb, b0, w1, b1, w2, b2, g, be, hn_ref):
    t = jnp.maximum(_dot(agg[...], wa[...])
                    + _dot(h[...], wb[...]) + b0[...], 0.0)
    t = jnp.maximum(_dot(t, w1[...]) + b1[...], 0.0)
    hn_ref[...] = h[...] + _ln(_dot(t, w2[...]) + b2[...], g[...], be[...])


def _dec_body(h, w0, b0, w1, b1, w2, b2, out_ref):
    t = jnp.maximum(_dot(h[...], w0[...]) + b0[...], 0.0)
    t = jnp.maximum(_dot(t, w1[...]) + b1[...], 0.0)
    out_ref[...] = _dot(t, w2[...]) + b2[...]


def _tc_call(body, nrows, block, in_specs_cols, weight_shapes, out_cols,
             n_out, out_dtypes=None):
    grid = nrows // block
    in_specs = ([_row_spec(block, c) for c in in_specs_cols]
                + [_full_spec(s) for s in weight_shapes])
    out_specs = [_row_spec(block, c) for c in out_cols]
    if out_dtypes is None:
        out_dtypes = [F32] * len(out_cols)
    out_shape = [jax.ShapeDtypeStruct((nrows, c), d)
                 for c, d in zip(out_cols, out_dtypes)]
    if n_out == 1:
        out_specs, out_shape = out_specs[0], out_shape[0]
    return pl.pallas_call(
        body,
        grid=(grid,),
        in_specs=in_specs,
        out_specs=out_specs,
        out_shape=out_shape,
        compiler_params=pltpu.CompilerParams(
            dimension_semantics=("arbitrary",)),
    )


_HALF = LATENT // NC
_NPS = N_NODES // NS


@functools.lru_cache(maxsize=None)
def _build_sc_gather():
    mesh = plsc.VectorSubcoreMesh(core_axis_name="c", subcore_axis_name="s")

    nk = (NCHUNK + NW - 1) // NW

    @functools.partial(
        pl.kernel,
        out_type=(jax.ShapeDtypeStruct((N_EDGES, LATENT), F32),
                  jax.ShapeDtypeStruct((N_EDGES, LATENT), F32)),
        mesh=mesh,
        scratch_types=[
            pltpu.VMEM((CH,), jnp.int32),
            pltpu.VMEM((CH,), jnp.int32),
            pltpu.VMEM((CH, LATENT), F32),
            pltpu.VMEM((CH, LATENT), F32),
            pltpu.SemaphoreType.DMA,
        ],
    )
    def sc_gather(ha_hbm, hb_hbm, dst_hbm, src_hbm, xi_hbm, xj_hbm,
                  idxd_v, idxs_v, bufa_v, bufb_v, sem):
        wid = lax.axis_index("s") * NC + lax.axis_index("c")

        def chunk(k, carry):
            c = k * NW + wid

            @pl.when(c < NCHUNK)
            def _():
                base = c * CH
                pltpu.sync_copy(dst_hbm.at[pl.ds(base, CH)], idxd_v)
                pltpu.sync_copy(src_hbm.at[pl.ds(base, CH)], idxs_v)
                a = pltpu.async_copy(ha_hbm.at[idxd_v], bufa_v, sem)
                b = pltpu.async_copy(hb_hbm.at[idxs_v], bufb_v, sem)
                a.wait()
                b.wait()
                pltpu.sync_copy(bufa_v, xi_hbm.at[pl.ds(base, CH)])
                pltpu.sync_copy(bufb_v, xj_hbm.at[pl.ds(base, CH)])
            return carry

        lax.fori_loop(0, nk, chunk, 0)

    return sc_gather


@functools.lru_cache(maxsize=None)
def _build_sc_permute():
    mesh = plsc.VectorSubcoreMesh(core_axis_name="c", subcore_axis_name="s")

    nk = (NCHUNK + NW - 1) // NW

    @functools.partial(
        pl.kernel,
        out_type=jax.ShapeDtypeStruct((N_EDGES, LATENT), F32),
        mesh=mesh,
        scratch_types=[
            pltpu.VMEM((CH,), jnp.int32),
            pltpu.VMEM((CH, LATENT), F32),
            pltpu.SemaphoreType.DMA,
        ],
    )
    def sc_permute(tab_hbm, perm_hbm, out_hbm, idx_v, buf_v, sem):
        wid = lax.axis_index("s") * NC + lax.axis_index("c")

        def chunk(k, carry):
            c = k * NW + wid

            @pl.when(c < NCHUNK)
            def _():
                base = c * CH
                pltpu.sync_copy(perm_hbm.at[pl.ds(base, CH)], idx_v)
                pltpu.async_copy(tab_hbm.at[idx_v], buf_v, sem).wait()
                pltpu.sync_copy(buf_v, out_hbm.at[pl.ds(base, CH)])
            return carry

        lax.fori_loop(0, nk, chunk, 0)

    return sc_permute


_CPB = NCHUNK // NS
_CREM = NCHUNK - _CPB * NS


@functools.lru_cache(maxsize=None)
def _build_sc_scatter():
    mesh = plsc.VectorSubcoreMesh(core_axis_name="c", subcore_axis_name="s")

    nzc = ACCR // SK

    @functools.partial(
        pl.kernel,
        out_type=jax.ShapeDtypeStruct((N_NODES, LATENT), F32),
        mesh=mesh,
        scratch_types=[
            pltpu.VMEM((CH,), jnp.int32),
            pltpu.VMEM((CH, _HALF), F32),
            pltpu.VMEM_SHARED((ACCR, _HALF), F32),
        ],
    )
    def sc_scatter(msg_hbm, dst_hbm, zeros_hbm, agg_hbm,
                   idx_v, rows_v, acc_sh):
        cid = lax.axis_index("c")
        sid = lax.axis_index("s")
        col0 = cid * _HALF
        for k in range((nzc + NS - 1) // NS):
            c = sid + k * NS

            @pl.when(c < nzc)
            def _():
                pltpu.sync_copy(zeros_hbm, acc_sh.at[pl.ds(c * SK, SK)])
        plsc.subcore_barrier()

        cbase = sid * _CPB + jnp.minimum(sid, _CREM)
        ccnt = _CPB + jnp.where(sid < _CREM, 1, 0)

        def chunk(k, carry):
            @pl.when(k < ccnt)
            def _():
                base = (cbase + k) * CH
                pltpu.sync_copy(dst_hbm.at[pl.ds(base, CH)], idx_v)
                pltpu.sync_copy(
                    msg_hbm.at[pl.ds(base, CH), pl.ds(col0, _HALF)],
                    rows_v)
                pltpu.sync_copy(rows_v, acc_sh.at[idx_v], add=True)
            return carry

        lax.fori_loop(0, _CPB + (1 if _CREM else 0), chunk, 0)
        plsc.subcore_barrier()
        for k in range((nzc + NS - 1) // NS):
            c = sid + k * NS

            @pl.when(c < nzc)
            def _():
                pltpu.sync_copy(
                    acc_sh.at[pl.ds(c * SK, SK)],
                    agg_hbm.at[pl.ds(c * SK, SK), pl.ds(col0, _HALF)])

    return sc_scatter


def _sc_gather(ha, hb, dst, src):
    return _build_sc_gather()(ha, hb, dst, src)


def _sc_permute(tab, perm):
    return _build_sc_permute()(tab, perm)


def _sc_scatter(msg, dst, zeros):
    return _build_sc_scatter()(msg, dst, zeros)


def _prep_mlp(ps):
    out = []
    for W, b in ps:
        out.append(W)
        out.append(b.reshape(1, -1))
    return out


def kernel(x, edge_index, edge_features, params):
    idx = edge_index.astype(jnp.int32)
    zeros = jnp.zeros((SK, _HALF), F32)
    perm = jnp.argsort(idx[1], stable=True).astype(jnp.int32)
    src = idx[0][perm]
    dst = idx[1][perm]

    blocks = params["blocks"]
    eW1 = [blk["edge_mlp"][0][0] for blk in blocks]
    eW1a = [w[:LATENT] for w in eW1]
    eW1b = [w[LATENT:2 * LATENT] for w in eW1]
    eW1c = [w[2 * LATENT:] for w in eW1]
    nW1 = [blk["node_mlp"][0][0] for blk in blocks]
    nW1a = [w[:LATENT] for w in nW1]
    nW1b = [w[LATENT:] for w in nW1]

    WS = (LATENT, LATENT)
    BS1 = (1, LATENT)

    ne = _prep_mlp(params["node_enc"])
    g, be = params["node_enc_ln"]
    node_enc = _tc_call(
        _node_enc_body, N_NODES, BN, [x.shape[1]],
        [(x.shape[1], LATENT), BS1, WS, BS1, WS, BS1, BS1, BS1, WS, WS],
        [LATENT, LATENT, LATENT], 3)
    h, ha, hb = node_enc(x, ne[0], ne[1], ne[2], ne[3], ne[4], ne[5],
                         g.reshape(1, -1), be.reshape(1, -1),
                         eW1a[0], eW1b[0])

    ee = _prep_mlp(params["edge_enc"])
    ge, bee = params["edge_enc_ln"]
    nin_e = edge_features.shape[1]
    edge_enc = _tc_call(
        _edge_enc_body, N_EDGES, BE, [nin_e],
        [(nin_e, LATENT), BS1, WS, BS1, WS, BS1, BS1, BS1],
        [LATENT], 1)
    e = edge_enc(edge_features, ee[0], ee[1], ee[2], ee[3], ee[4], ee[5],
                 ge.reshape(1, -1), bee.reshape(1, -1))
    e = _sc_permute(e, perm)

    edge_step = _tc_call(
        _edge_step_body, N_EDGES, BE, [LATENT, LATENT, LATENT],
        [WS, BS1, WS, BS1, WS, BS1, BS1, BS1],
        [LATENT, LATENT], 2)
    node_step = _tc_call(
        _node_step_body, N_NODES, BN, [LATENT, LATENT],
        [WS, WS, BS1, WS, BS1, WS, BS1, BS1, BS1, WS, WS],
        [LATENT, LATENT, LATENT], 3)
    node_last = _tc_call(
        _node_last_body, N_NODES, BN, [LATENT, LATENT],
        [WS, WS, BS1, WS, BS1, WS, BS1, BS1, BS1],
        [LATENT], 1)

    nsteps = len(blocks)
    for s in range(nsteps):
        blk = blocks[s]
        em = _prep_mlp(blk["edge_mlp"])
        nm = _prep_mlp(blk["node_mlp"])
        eg, eb = blk["edge_ln"]
        ng, nb = blk["node_ln"]

        xi, xj = _sc_gather(ha, hb, dst, src)
        msg, e = edge_step(xi, xj, e, eW1c[s], em[1], em[2], em[3],
                           em[4], em[5], eg.reshape(1, -1), eb.reshape(1, -1))
        agg = _sc_scatter(msg, dst, zeros)
        if s + 1 < nsteps:
            h, ha, hb = node_step(agg, h, nW1a[s], nW1b[s], nm[1], nm[2],
                                  nm[3], nm[4], nm[5], ng.reshape(1, -1),
                                  nb.reshape(1, -1), eW1a[s + 1], eW1b[s + 1])
        else:
            h = node_last(agg, h, nW1a[s], nW1b[s], nm[1], nm[2], nm[3],
                          nm[4], nm[5], ng.reshape(1, -1), nb.reshape(1, -1))

    dm = _prep_mlp(params["dec"])
    nout = dm[4].shape[1]
    w2p = jnp.zeros((LATENT, 128), F32).at[:, :nout].set(dm[4])
    b2p = jnp.zeros((1, 128), F32).at[:, :nout].set(dm[5])
    dec = _tc_call(
        _dec_body, N_NODES, BN, [LATENT],
        [WS, BS1, WS, BS1, (LATENT, 128), (1, 128)],
        [128], 1)
    out = dec(h, dm[0], dm[1], dm[2], dm[3], w2p, b2p)
    return out[:, :nout]
